# baseline (speedup 1.0000x reference)
import functools

import jax
import jax.numpy as jnp
import numpy as np
from jax.experimental import pallas as pl
from jax.experimental.pallas import tpu as pltpu
from jax.experimental.pallas import tpu_sc as plsc

_B, _N = 8, 32768
_S = 64
_STEM = 64
_TOK = 128
_RADII = (0.04, 0.08)
_KS = (16, 32)
_TBL_D = 128
_ROWS_PER_B = _S + _S * _KS[0] + _S * _KS[1]
_G = _B * _ROWS_PER_B
_C = 2048
_NCH = _N // _C

_INTERPRET = False


def _split_dot(x, m):
    xh = x.astype(jnp.bfloat16).astype(jnp.float32)
    xl = x - xh
    return (jnp.dot(xh, m, preferred_element_type=jnp.float32)
            + jnp.dot(xl, m, preferred_element_type=jnp.float32))


def _ln(x, g, beta):
    d = x.shape[-1]
    io = jax.lax.broadcasted_iota(jnp.int32, (d, d), 0)
    ones_d = jnp.where(io >= 0, np.float32(1.0), np.float32(0.0))
    inv = np.float32(1.0 / d)
    mu = _split_dot(x, ones_d) * inv
    ex2 = _split_dot(x * x, ones_d) * inv
    rs = jax.lax.rsqrt(ex2 - mu * mu + 1e-5)
    return (x - mu) * rs * g + beta


def _gelu(x):
    return x * 0.5 * (1.0 + jax.lax.erf(x / np.sqrt(2.0).astype(np.float32)))


def _stem_body(x_ref, w1, b1, g1, be1, w2, b2, g2, be2, out_ref):
    x6 = x_ref[...]
    x = jnp.concatenate(
        [x6, jnp.zeros((x6.shape[0], 2), jnp.float32)], axis=1)
    h = jnp.dot(x, w1[...], preferred_element_type=jnp.float32) + b1[...]
    h = _gelu(_ln(h, g1[...], be1[...]))
    h = jnp.dot(h, w2[...], preferred_element_type=jnp.float32) + b2[...]
    h = _gelu(_ln(h, g2[...], be2[...]))
    xyz = x[:, 0:3]
    pad = jnp.zeros((x.shape[0], _TBL_D - _STEM - 3), jnp.float32)
    out_ref[...] = jnp.concatenate([h, xyz, pad], axis=1)


def _run_stem(xpad, sp):
    blk = 4096
    grid = (_B * _N // blk,)
    full = lambda a: pl.BlockSpec(a.shape, lambda i: (0,) * a.ndim)
    params = []
    specs = [pl.BlockSpec((blk, 6), lambda i: (i, 0))]
    for layer in sp:
        for arr in layer:
            params.append(arr)
            specs.append(full(arr))
    return pl.pallas_call(
        _stem_body,
        grid=grid,
        in_specs=specs,
        out_specs=pl.BlockSpec((blk, _TBL_D), lambda i: (i, 0)),
        out_shape=jax.ShapeDtypeStruct((_B * _N, _TBL_D), jnp.float32),
        interpret=_INTERPRET,
    )(xpad, *params)


def _fps_body(x_ref, y_ref, z_ref, cidx_ref, cen_ref, dist_ref):
    shp = (_B, _N // 128, 128)
    ax = (1, 2)
    flat = (jax.lax.broadcasted_iota(jnp.int32, shp, 1) * 128
            + jax.lax.broadcasted_iota(jnp.int32, shp, 2))
    ii64 = jax.lax.broadcasted_iota(jnp.int32, (_B, _S), 1)
    cc_i = jax.lax.broadcasted_iota(jnp.int32, (_B, 3, _S), 2)
    dist_ref[...] = jnp.full(shp, 1e10, jnp.float32)

    def body(i, carry):
        far, ci, cc = carry
        ci = jnp.where(ii64 == i, far[:, :, 0], ci)
        sel = flat == far
        cx = jnp.sum(jnp.where(sel, x_ref[...], 0.0), axis=ax, keepdims=True)
        cy = jnp.sum(jnp.where(sel, y_ref[...], 0.0), axis=ax, keepdims=True)
        cz = jnp.sum(jnp.where(sel, z_ref[...], 0.0), axis=ax, keepdims=True)
        coords = jnp.concatenate(
            [cx[:, :, 0], cy[:, :, 0], cz[:, :, 0]],
            axis=1)[:, :, None]
        cc = jnp.where(cc_i == i, coords, cc)
        dx = x_ref[...] - cx
        dy = y_ref[...] - cy
        dz = z_ref[...] - cz
        d = dx * dx + dy * dy + dz * dz
        dn = jnp.minimum(dist_ref[...], d)
        dist_ref[...] = dn
        m = jnp.max(dn, axis=ax, keepdims=True)
        far = jnp.min(jnp.where(dn == m, flat, jnp.int32(_N)),
                      axis=ax, keepdims=True)
        return far, ci, cc

    far0 = jnp.zeros((_B, 1, 1), jnp.int32)
    ci0 = jnp.zeros((_B, _S), jnp.int32)
    cc0 = jnp.zeros((_B, 3, _S), jnp.float32)
    _, ci, cc = jax.lax.fori_loop(0, _S, body, (far0, ci0, cc0))
    cidx_ref[...] = ci
    cen_ref[...] = cc


def _run_fps(x3, y3, z3):
    return pl.pallas_call(
        _fps_body,
        out_shape=(jax.ShapeDtypeStruct((_B, _S), jnp.int32),
                   jax.ShapeDtypeStruct((_B, 3, _S), jnp.float32)),
        scratch_shapes=[pltpu.VMEM((_B, _N // 128, 128), jnp.float32)],
        interpret=_INTERPRET,
    )(x3, y3, z3)


def _ball_body(x_ref, y_ref, z_ref, cx_ref, cy_ref, cz_ref, cidx_ref,
               oall_ref, acc1, acc2, cnt1, cnt2):
    b = pl.program_id(0)
    j = pl.program_id(1)

    @pl.when(j == 0)
    def _():
        acc1[...] = jnp.zeros_like(acc1)
        acc2[...] = jnp.zeros_like(acc2)
        cnt1[...] = jnp.zeros_like(cnt1)
        cnt2[...] = jnp.zeros_like(cnt2)

    ngrp = _C // 128
    X = x_ref[0, 0][None]
    dx = cx_ref[0][:, :, None] - X
    dy = cy_ref[0][:, :, None] - y_ref[0, 0][None]
    dz = cz_ref[0][:, :, None] - z_ref[0, 0][None]
    d2 = dx * dx + dy * dy + dz * dz

    in1 = d2 <= np.float32(_RADII[0] * _RADII[0])
    in2 = d2 <= np.float32(_RADII[1] * _RADII[1])
    mp = (jnp.where(in1, 1 << 13, 0) + jnp.where(in2, 1, 0))
    sh = 1
    while sh < 128:
        z = jnp.zeros((_S, ngrp, sh), jnp.int32)
        mp = mp + jnp.concatenate([z, mp[:, :, : 128 - sh]], axis=2)
        sh *= 2
    lane3 = jax.lax.broadcasted_iota(jnp.int32, (_S, ngrp, 128), 2)
    goff = (j * _C
            + jax.lax.broadcasted_iota(jnp.int32, (1, ngrp), 1) * 128)
    L = 4

    for inm, glr, K, acc, cnt in (
            (in1, mp >> 13, _KS[0], acc1, cnt1),
            (in2, mp & 8191, _KS[1], acc2, cnt2)):
        gcnt = glr[:, :, 127]
        inc = gcnt
        sh = 1
        while sh < ngrp:
            z = jnp.zeros((_S, sh), jnp.int32)
            inc = inc + jnp.concatenate([z, inc[:, : ngrp - sh]], axis=1)
            sh *= 2
        ex = inc - gcnt
        total = inc[:, ngrp - 1:ngrp]
        cold = cnt[...]
        overflow = jnp.any(gcnt > L)

        @pl.when(overflow)
        def _(inm=inm, glr=glr, K=K, acc=acc, cold=cold, ex=ex):
            rr = jnp.where(inm, glr + ex[:, :, None] + cold[:, :, None], 0)
            jg3 = goff[:, :, None] + lane3
            contribs = [jnp.sum(jnp.where(rr == k + 1, jg3, 0),
                                axis=(1, 2), keepdims=True)[:, :, 0]
                        for k in range(K)]
            acc[...] += jnp.concatenate(contribs, axis=1)

        @pl.when(jnp.logical_not(overflow))
        def _(inm=inm, glr=glr, K=K, acc=acc, cold=cold, ex=ex, gcnt=gcnt):
            vals, slots = [], []
            for t in range(L):
                gv = jnp.sum(jnp.where(jnp.logical_and(inm, glr == t + 1),
                                       lane3, 0), axis=2)
                vals.append(gv + goff)
                slots.append(jnp.where(gcnt > t, ex + cold + (t + 1), 0))
            valsc = jnp.concatenate(vals, axis=1)
            slotc = jnp.concatenate(slots, axis=1)
            contribs = [jnp.sum(jnp.where(slotc == k + 1, valsc, 0),
                                axis=1, keepdims=True) for k in range(K)]
            acc[...] += jnp.concatenate(contribs, axis=1)

        cnt[...] = cold + total

    @pl.when(j == _NCH - 1)
    def _():
        off = b * _N
        for K, acc, cnt, lo in ((_KS[0], acc1, cnt1, 0),
                                (_KS[1], acc2, cnt2, _KS[0])):
            ids = acc[...]
            kk = jax.lax.broadcasted_iota(jnp.int32, (_S, K), 1)
            padded = jnp.where(kk < cnt[...], ids, ids[:, 0:1]) + off
            oall_ref[0, lo:lo + K, :] = jnp.transpose(padded, (1, 0))
        oall_ref[0, 48:49, :] = cidx_ref[0] + off


def _run_ball(x2, y2, z2, cenx, ceny, cenz, cidx3):
    cspec = pl.BlockSpec((1, _S, 1), lambda b, j: (b, 0, 0))
    return pl.pallas_call(
        _ball_body,
        grid=(_B, _NCH),
        in_specs=[
            pl.BlockSpec((1, 1, _C // 128, 128), lambda b, j: (b, j, 0, 0)),
            pl.BlockSpec((1, 1, _C // 128, 128), lambda b, j: (b, j, 0, 0)),
            pl.BlockSpec((1, 1, _C // 128, 128), lambda b, j: (b, j, 0, 0)),
            cspec, cspec, cspec,
            pl.BlockSpec((1, 1, _S), lambda b, j: (b, 0, 0)),
        ],
        out_specs=pl.BlockSpec((1, 49, _S), lambda b, j: (b, 0, 0)),
        out_shape=jax.ShapeDtypeStruct((_B, 49, _S), jnp.int32),
        scratch_shapes=[
            pltpu.VMEM((_S, _KS[0]), jnp.int32),
            pltpu.VMEM((_S, _KS[1]), jnp.int32),
            pltpu.VMEM((_S, 1), jnp.int32),
            pltpu.VMEM((_S, 1), jnp.int32),
        ],
        interpret=_INTERPRET,
    )(x2, y2, z2, cenx, ceny, cenz, cidx3)


def _gather_rows(table, gidx):
    nw = 32
    bpw = _G // nw
    mesh = plsc.VectorSubcoreMesh(core_axis_name="c", subcore_axis_name="s")

    @functools.partial(
        pl.kernel,
        mesh=mesh,
        out_type=jax.ShapeDtypeStruct((_G, _TBL_D), jnp.float32),
        scratch_types=[
            pltpu.VMEM((bpw,), jnp.int32),
            pltpu.VMEM((bpw, _TBL_D), jnp.float32),
            pltpu.SemaphoreType.DMA,
        ],
    )
    def k(table_hbm, idx_hbm, out_hbm, idx_v, rows_v, sem):
        wid = jax.lax.axis_index("s") * 2 + jax.lax.axis_index("c")
        base = wid * bpw
        pltpu.sync_copy(idx_hbm.at[pl.ds(base, bpw)], idx_v)
        pltpu.async_copy(table_hbm.at[idx_v], rows_v, sem).wait()
        pltpu.sync_copy(rows_v, out_hbm.at[pl.ds(base, bpw)])

    return k(table, gidx)


def _sin3(x, dim):
    d_axis = dim // 3
    half = d_axis // 2
    scale = np.float32(-np.log(10000.0) / max(half - 1, 1))
    col = jax.lax.broadcasted_iota(jnp.int32, (3, dim), 1)
    row = jax.lax.broadcasted_iota(jnp.int32, (3, dim), 0)
    freqs = jnp.exp((col % half).astype(jnp.float32) * scale)
    proj = jnp.where(col // d_axis == row, freqs, np.float32(0.0))
    ph = proj.astype(jnp.bfloat16).astype(jnp.float32)
    ang = _split_dot(x, ph) + jnp.dot(
        x, proj - ph, preferred_element_type=jnp.float32)
    colf = jax.lax.broadcasted_iota(jnp.int32, (x.shape[0], dim), 1)
    off = jnp.where((colf % d_axis) >= half, np.float32(np.pi / 2),
                    np.float32(0.0))
    return jnp.sin(ang + off)


def _token_body(g_ref, ce_ref,
                e10w, e10b, e10g, e10be, e11w, e11b, e11g, e11be,
                e20w, e20b, e20g, e20be, e21w, e21b, e21g, e21be,
                p0w, p0b, p0g, p0be, p1w, p1b, p1g, p1be,
                tok_ref):
    gg = g_ref[0]
    cen = ce_ref[0]
    cf = gg[3072:3136, 0:_STEM]

    def scale(lo, hi, K, w0, b0, g0, be0, w1, b1, g1, be1):
        gf = gg[lo:hi, 0:_STEM]
        gx = gg[lo:hi, _STEM:_STEM + 3]
        rel = gx - jnp.concatenate([cen] * K, axis=0)
        rpe = _sin3(rel, 24)
        pad = jnp.zeros((hi - lo, 5), jnp.float32)
        gi = jnp.concatenate([gf, rel, rpe, pad], axis=1)
        h = jnp.dot(gi, w0[...], preferred_element_type=jnp.float32) + b0[...]
        h = _gelu(_ln(h, g0[...], be0[...]))
        h = jnp.dot(h, w1[...], preferred_element_type=jnp.float32) + b1[...]
        h = _ln(h, g1[...], be1[...])
        mx = h[0:_S]
        for kk in range(1, K):
            mx = jnp.maximum(mx, h[kk * _S:(kk + 1) * _S])
        return mx

    lo1, hi1 = 0, _S * _KS[0]
    lo2, hi2 = hi1, hi1 + _S * _KS[1]
    mx1 = scale(lo1, hi1, _KS[0], e10w, e10b, e10g, e10be,
                e11w, e11b, e11g, e11be)
    mx2 = scale(lo2, hi2, _KS[1], e20w, e20b, e20g, e20be,
                e21w, e21b, e21g, e21be)
    cpos = _sin3(cen, 96)
    t = jnp.concatenate([cf, mx1, mx2, cpos], axis=1)
    t = jnp.dot(t, p0w[...], preferred_element_type=jnp.float32) + p0b[...]
    t = _gelu(_ln(t, p0g[...], p0be[...]))
    t = jnp.dot(t, p1w[...], preferred_element_type=jnp.float32) + p1b[...]
    t = _ln(t, p1g[...], p1be[...])
    tok_ref[...] = t[None]


def _run_token(g3, cen, flat_params):
    full = lambda a: pl.BlockSpec(a.shape, lambda b: (0,) * a.ndim)
    specs = [
        pl.BlockSpec((1, _ROWS_PER_B, _TBL_D), lambda b: (b, 0, 0)),
        pl.BlockSpec((1, _S, 3), lambda b: (b, 0, 0)),
    ] + [full(a) for a in flat_params]
    return pl.pallas_call(
        _token_body,
        grid=(_B,),
        in_specs=specs,
        out_specs=pl.BlockSpec((1, _S, _TOK), lambda b: (b, 0, 0)),
        out_shape=jax.ShapeDtypeStruct((_B, _S, _TOK), jnp.float32),
        interpret=_INTERPRET,
    )(g3, cen, *flat_params)


def _prep_layer(p, pad_rows=None):
    w = p['W']
    if pad_rows is not None and w.shape[0] < pad_rows:
        w = jnp.pad(w, ((0, pad_rows - w.shape[0]), (0, 0)))
    return (w, p['b'][None, :], p['g'][None, :], p['beta'][None, :])


def kernel(pointcloud, params):
    pc = pointcloud.reshape(_B * _N, 6)

    x2 = pointcloud[..., 0]
    y2 = pointcloud[..., 1]
    z2 = pointcloud[..., 2]
    x4 = x2.reshape(_B, _NCH, _C // 128, 128)
    y4 = y2.reshape(_B, _NCH, _C // 128, 128)
    z4 = z2.reshape(_B, _NCH, _C // 128, 128)

    stem_params = [_prep_layer(params['stem'][0], pad_rows=8),
                   _prep_layer(params['stem'][1])]
    table = _run_stem(pc, stem_params)

    cidx, cc = _run_fps(x2.reshape(_B, _N // 128, 128),
                        y2.reshape(_B, _N // 128, 128),
                        z2.reshape(_B, _N // 128, 128))
    cenx = cc[:, 0, :, None]
    ceny = cc[:, 1, :, None]
    cenz = cc[:, 2, :, None]
    centers = jnp.transpose(cc, (0, 2, 1))

    oall = _run_ball(x4, y4, z4, cenx, ceny, cenz, cidx[:, None, :])

    gidx = oall.reshape(_G)

    rows = _gather_rows(table, gidx)
    g3 = rows.reshape(_B, _ROWS_PER_B, _TBL_D)

    flat_params = []
    for layer_i, layer in enumerate(params['enc'][0] + params['enc'][1]):
        flat_params.extend(_prep_layer(layer, pad_rows=96 if layer_i % 2 == 0 else None))
    for layer in params['proj']:
        flat_params.extend(_prep_layer(layer))

    tok = _run_token(g3, centers, flat_params)
    return tok, centers


# speedup vs baseline: 1.6957x; 1.6957x over previous
import functools

import jax
import jax.numpy as jnp
import numpy as np
from jax.experimental import pallas as pl
from jax.experimental.pallas import tpu as pltpu
from jax.experimental.pallas import tpu_sc as plsc

_B, _N = 8, 32768
_S = 64
_STEM = 64
_TOK = 128
_RADII = (0.04, 0.08)
_KS = (16, 32)
_TBL_D = 128
_ROWS_PER_B = _S + _S * _KS[0] + _S * _KS[1]
_G = _B * _ROWS_PER_B
_C = 4096
_NCH = _N // _C

_INTERPRET = False


def _split_dot(x, m):
    xh = x.astype(jnp.bfloat16).astype(jnp.float32)
    xl = x - xh
    return (jnp.dot(xh, m, preferred_element_type=jnp.float32)
            + jnp.dot(xl, m, preferred_element_type=jnp.float32))


def _ln(x, g, beta):
    d = x.shape[-1]
    io = jax.lax.broadcasted_iota(jnp.int32, (d, d), 0)
    ones_d = jnp.where(io >= 0, np.float32(1.0), np.float32(0.0))
    inv = np.float32(1.0 / d)
    mu = _split_dot(x, ones_d) * inv
    ex2 = _split_dot(x * x, ones_d) * inv
    rs = jax.lax.rsqrt(ex2 - mu * mu + 1e-5)
    return (x - mu) * rs * g + beta


def _gelu(x):
    return x * 0.5 * (1.0 + jax.lax.erf(x / np.sqrt(2.0).astype(np.float32)))


def _stem_body(x_ref, w1, b1, g1, be1, w2, b2, g2, be2, out_ref):
    x6 = x_ref[...]
    x = jnp.concatenate(
        [x6, jnp.zeros((x6.shape[0], 2), jnp.float32)], axis=1)
    h = jnp.dot(x, w1[...], preferred_element_type=jnp.float32) + b1[...]
    h = _gelu(_ln(h, g1[...], be1[...]))
    h = jnp.dot(h, w2[...], preferred_element_type=jnp.float32) + b2[...]
    h = _gelu(_ln(h, g2[...], be2[...]))
    xyz = x[:, 0:3]
    pad = jnp.zeros((x.shape[0], _TBL_D - _STEM - 3), jnp.float32)
    out_ref[...] = jnp.concatenate([h, xyz, pad], axis=1)


def _run_stem(xpad, sp):
    blk = 4096
    grid = (_B * _N // blk,)
    full = lambda a: pl.BlockSpec(a.shape, lambda i: (0,) * a.ndim)
    params = []
    specs = [pl.BlockSpec((blk, 6), lambda i: (i, 0))]
    for layer in sp:
        for arr in layer:
            params.append(arr)
            specs.append(full(arr))
    return pl.pallas_call(
        _stem_body,
        grid=grid,
        in_specs=specs,
        out_specs=pl.BlockSpec((blk, _TBL_D), lambda i: (i, 0)),
        out_shape=jax.ShapeDtypeStruct((_B * _N, _TBL_D), jnp.float32),
        interpret=_INTERPRET,
    )(xpad, *params)


def _fps_body(x_ref, y_ref, z_ref, cidx_ref, cen_ref, dist_ref):
    shp = (_B, _N // 128, 128)
    ax = (1, 2)
    flat = (jax.lax.broadcasted_iota(jnp.int32, shp, 1) * 128
            + jax.lax.broadcasted_iota(jnp.int32, shp, 2))
    ii64 = jax.lax.broadcasted_iota(jnp.int32, (_B, _S), 1)
    cc_i = jax.lax.broadcasted_iota(jnp.int32, (_B, 3, _S), 2)
    dist_ref[...] = jnp.full(shp, 1e10, jnp.float32)

    def body(i, carry):
        far, ci, cc = carry
        ci = jnp.where(ii64 == i, far[:, :, 0], ci)
        sel = flat == far
        cx = jnp.sum(jnp.where(sel, x_ref[...], 0.0), axis=ax, keepdims=True)
        cy = jnp.sum(jnp.where(sel, y_ref[...], 0.0), axis=ax, keepdims=True)
        cz = jnp.sum(jnp.where(sel, z_ref[...], 0.0), axis=ax, keepdims=True)
        coords = jnp.concatenate(
            [cx[:, :, 0], cy[:, :, 0], cz[:, :, 0]],
            axis=1)[:, :, None]
        cc = jnp.where(cc_i == i, coords, cc)
        dx = x_ref[...] - cx
        dy = y_ref[...] - cy
        dz = z_ref[...] - cz
        d = dx * dx + dy * dy + dz * dz
        dn = jnp.minimum(dist_ref[...], d)
        dist_ref[...] = dn
        m = jnp.max(dn, axis=ax, keepdims=True)
        far = jnp.min(jnp.where(dn == m, flat, jnp.int32(_N)),
                      axis=ax, keepdims=True)
        return far, ci, cc

    far0 = jnp.zeros((_B, 1, 1), jnp.int32)
    ci0 = jnp.zeros((_B, _S), jnp.int32)
    cc0 = jnp.zeros((_B, 3, _S), jnp.float32)
    _, ci, cc = jax.lax.fori_loop(0, _S, body, (far0, ci0, cc0))
    cidx_ref[...] = ci
    cen_ref[...] = cc


def _run_fps(x3, y3, z3):
    return pl.pallas_call(
        _fps_body,
        out_shape=(jax.ShapeDtypeStruct((_B, _S), jnp.int32),
                   jax.ShapeDtypeStruct((_B, 3, _S), jnp.float32)),
        scratch_shapes=[pltpu.VMEM((_B, _N // 128, 128), jnp.float32)],
        interpret=_INTERPRET,
    )(x3, y3, z3)


def _ball_body(x_ref, y_ref, z_ref, cx_ref, cy_ref, cz_ref, cidx_ref,
               oall_ref, acc1, acc2, cnt1, cnt2):
    b = pl.program_id(0)
    j = pl.program_id(1)

    @pl.when(j == 0)
    def _():
        acc1[...] = jnp.zeros_like(acc1)
        acc2[...] = jnp.zeros_like(acc2)
        cnt1[...] = jnp.zeros_like(cnt1)
        cnt2[...] = jnp.zeros_like(cnt2)

    dx = cx_ref[0] - x_ref[0, 0]
    dy = cy_ref[0] - y_ref[0, 0]
    dz = cz_ref[0] - z_ref[0, 0]
    d2 = dx * dx + dy * dy + dz * dz
    jg = (j * _C + jax.lax.broadcasted_iota(jnp.int32, (1, _C), 1))
    jgb = jnp.broadcast_to(jg, (_S, _C))

    in1 = d2 <= np.float32(_RADII[0] * _RADII[0])
    in2 = d2 <= np.float32(_RADII[1] * _RADII[1])
    mp = (jnp.where(in1, 1 << 13, 0) + jnp.where(in2, 1, 0))
    sh = 1
    while sh < _C:
        z = jnp.zeros((_S, sh), jnp.int32)
        mp = mp + jnp.concatenate([z, mp[:, : _C - sh]], axis=1)
        sh *= 2
    for inm, rank, K, acc, cnt in (
            (in1, mp >> 13, _KS[0], acc1, cnt1),
            (in2, mp & 8191, _KS[1], acc2, cnt2)):
        cold = cnt[...]
        rr = jnp.where(inm, rank + cold, 0)
        contribs = [jnp.sum(jnp.where(rr == k + 1, jgb, 0),
                            axis=1, keepdims=True) for k in range(K)]
        acc[...] += jnp.concatenate(contribs, axis=1)
        cnt[...] = cold + rank[:, _C - 1:_C]

    @pl.when(j == _NCH - 1)
    def _():
        off = b * _N
        for K, acc, cnt, lo in ((_KS[0], acc1, cnt1, 0),
                                (_KS[1], acc2, cnt2, _KS[0])):
            ids = acc[...]
            kk = jax.lax.broadcasted_iota(jnp.int32, (_S, K), 1)
            padded = jnp.where(kk < cnt[...], ids, ids[:, 0:1]) + off
            oall_ref[0, lo:lo + K, :] = jnp.transpose(padded, (1, 0))
        oall_ref[0, 48:49, :] = cidx_ref[0] + off


def _run_ball(x2, y2, z2, cenx, ceny, cenz, cidx3):
    cspec = pl.BlockSpec((1, _S, 1), lambda b, j: (b, 0, 0))
    return pl.pallas_call(
        _ball_body,
        grid=(_B, _NCH),
        in_specs=[
            pl.BlockSpec((1, 1, 1, _C), lambda b, j: (b, j, 0, 0)),
            pl.BlockSpec((1, 1, 1, _C), lambda b, j: (b, j, 0, 0)),
            pl.BlockSpec((1, 1, 1, _C), lambda b, j: (b, j, 0, 0)),
            cspec, cspec, cspec,
            pl.BlockSpec((1, 1, _S), lambda b, j: (b, 0, 0)),
        ],
        out_specs=pl.BlockSpec((1, 49, _S), lambda b, j: (b, 0, 0)),
        out_shape=jax.ShapeDtypeStruct((_B, 49, _S), jnp.int32),
        scratch_shapes=[
            pltpu.VMEM((_S, _KS[0]), jnp.int32),
            pltpu.VMEM((_S, _KS[1]), jnp.int32),
            pltpu.VMEM((_S, 1), jnp.int32),
            pltpu.VMEM((_S, 1), jnp.int32),
        ],
        interpret=_INTERPRET,
    )(x2, y2, z2, cenx, ceny, cenz, cidx3)


def _gather_rows(table, gidx):
    nw = 32
    bpw = _G // nw
    mesh = plsc.VectorSubcoreMesh(core_axis_name="c", subcore_axis_name="s")

    @functools.partial(
        pl.kernel,
        mesh=mesh,
        out_type=jax.ShapeDtypeStruct((_G, _TBL_D), jnp.float32),
        scratch_types=[
            pltpu.VMEM((bpw,), jnp.int32),
            pltpu.VMEM((bpw, _TBL_D), jnp.float32),
            pltpu.SemaphoreType.DMA,
        ],
    )
    def k(table_hbm, idx_hbm, out_hbm, idx_v, rows_v, sem):
        wid = jax.lax.axis_index("s") * 2 + jax.lax.axis_index("c")
        base = wid * bpw
        pltpu.sync_copy(idx_hbm.at[pl.ds(base, bpw)], idx_v)
        pltpu.async_copy(table_hbm.at[idx_v], rows_v, sem).wait()
        pltpu.sync_copy(rows_v, out_hbm.at[pl.ds(base, bpw)])

    return k(table, gidx)


def _sin3(x, dim):
    d_axis = dim // 3
    half = d_axis // 2
    scale = np.float32(-np.log(10000.0) / max(half - 1, 1))
    col = jax.lax.broadcasted_iota(jnp.int32, (3, dim), 1)
    row = jax.lax.broadcasted_iota(jnp.int32, (3, dim), 0)
    freqs = jnp.exp((col % half).astype(jnp.float32) * scale)
    proj = jnp.where(col // d_axis == row, freqs, np.float32(0.0))
    ph = proj.astype(jnp.bfloat16).astype(jnp.float32)
    ang = _split_dot(x, ph) + jnp.dot(
        x, proj - ph, preferred_element_type=jnp.float32)
    colf = jax.lax.broadcasted_iota(jnp.int32, (x.shape[0], dim), 1)
    off = jnp.where((colf % d_axis) >= half, np.float32(np.pi / 2),
                    np.float32(0.0))
    return jnp.sin(ang + off)


def _token_body(g_ref, ce_ref,
                e10w, e10b, e10g, e10be, e11w, e11b, e11g, e11be,
                e20w, e20b, e20g, e20be, e21w, e21b, e21g, e21be,
                p0w, p0b, p0g, p0be, p1w, p1b, p1g, p1be,
                tok_ref):
    gg = g_ref[0]
    cen = ce_ref[0]
    cf = gg[3072:3136, 0:_STEM]

    def scale(lo, hi, K, w0, b0, g0, be0, w1, b1, g1, be1):
        gf = gg[lo:hi, 0:_STEM]
        gx = gg[lo:hi, _STEM:_STEM + 3]
        rel = gx - jnp.concatenate([cen] * K, axis=0)
        rpe = _sin3(rel, 24)
        pad = jnp.zeros((hi - lo, 5), jnp.float32)
        gi = jnp.concatenate([gf, rel, rpe, pad], axis=1)
        h = jnp.dot(gi, w0[...], preferred_element_type=jnp.float32) + b0[...]
        h = _gelu(_ln(h, g0[...], be0[...]))
        h = jnp.dot(h, w1[...], preferred_element_type=jnp.float32) + b1[...]
        h = _ln(h, g1[...], be1[...])
        mx = h[0:_S]
        for kk in range(1, K):
            mx = jnp.maximum(mx, h[kk * _S:(kk + 1) * _S])
        return mx

    lo1, hi1 = 0, _S * _KS[0]
    lo2, hi2 = hi1, hi1 + _S * _KS[1]
    mx1 = scale(lo1, hi1, _KS[0], e10w, e10b, e10g, e10be,
                e11w, e11b, e11g, e11be)
    mx2 = scale(lo2, hi2, _KS[1], e20w, e20b, e20g, e20be,
                e21w, e21b, e21g, e21be)
    cpos = _sin3(cen, 96)
    t = jnp.concatenate([cf, mx1, mx2, cpos], axis=1)
    t = jnp.dot(t, p0w[...], preferred_element_type=jnp.float32) + p0b[...]
    t = _gelu(_ln(t, p0g[...], p0be[...]))
    t = jnp.dot(t, p1w[...], preferred_element_type=jnp.float32) + p1b[...]
    t = _ln(t, p1g[...], p1be[...])
    tok_ref[...] = t[None]


def _run_token(g3, cen, flat_params):
    full = lambda a: pl.BlockSpec(a.shape, lambda b: (0,) * a.ndim)
    specs = [
        pl.BlockSpec((1, _ROWS_PER_B, _TBL_D), lambda b: (b, 0, 0)),
        pl.BlockSpec((1, _S, 3), lambda b: (b, 0, 0)),
    ] + [full(a) for a in flat_params]
    return pl.pallas_call(
        _token_body,
        grid=(_B,),
        in_specs=specs,
        out_specs=pl.BlockSpec((1, _S, _TOK), lambda b: (b, 0, 0)),
        out_shape=jax.ShapeDtypeStruct((_B, _S, _TOK), jnp.float32),
        interpret=_INTERPRET,
    )(g3, cen, *flat_params)


def _prep_layer(p, pad_rows=None):
    w = p['W']
    if pad_rows is not None and w.shape[0] < pad_rows:
        w = jnp.pad(w, ((0, pad_rows - w.shape[0]), (0, 0)))
    return (w, p['b'][None, :], p['g'][None, :], p['beta'][None, :])


def kernel(pointcloud, params):
    pc = pointcloud.reshape(_B * _N, 6)

    x2 = pointcloud[..., 0]
    y2 = pointcloud[..., 1]
    z2 = pointcloud[..., 2]
    x4 = x2.reshape(_B, _NCH, 1, _C)
    y4 = y2.reshape(_B, _NCH, 1, _C)
    z4 = z2.reshape(_B, _NCH, 1, _C)

    stem_params = [_prep_layer(params['stem'][0], pad_rows=8),
                   _prep_layer(params['stem'][1])]
    table = _run_stem(pc, stem_params)

    cidx, cc = _run_fps(x2.reshape(_B, _N // 128, 128),
                        y2.reshape(_B, _N // 128, 128),
                        z2.reshape(_B, _N // 128, 128))
    cenx = cc[:, 0, :, None]
    ceny = cc[:, 1, :, None]
    cenz = cc[:, 2, :, None]
    centers = jnp.transpose(cc, (0, 2, 1))

    oall = _run_ball(x4, y4, z4, cenx, ceny, cenz, cidx[:, None, :])

    gidx = oall.reshape(_G)

    rows = _gather_rows(table, gidx)
    g3 = rows.reshape(_B, _ROWS_PER_B, _TBL_D)

    flat_params = []
    for layer_i, layer in enumerate(params['enc'][0] + params['enc'][1]):
        flat_params.extend(_prep_layer(layer, pad_rows=96 if layer_i % 2 == 0 else None))
    for layer in params['proj']:
        flat_params.extend(_prep_layer(layer))

    tok = _run_token(g3, centers, flat_params)
    return tok, centers


# speedup vs baseline: 1.7242x; 1.0168x over previous
import functools

import jax
import jax.numpy as jnp
import numpy as np
from jax.experimental import pallas as pl
from jax.experimental.pallas import tpu as pltpu
from jax.experimental.pallas import tpu_sc as plsc

_B, _N = 8, 32768
_S = 64
_STEM = 64
_TOK = 128
_RADII = (0.04, 0.08)
_KS = (16, 32)
_TBL_D = 128
_ROWS_PER_B = _S + _S * _KS[0] + _S * _KS[1]
_G = _B * _ROWS_PER_B
_C = 8192
_NCH = _N // _C

_INTERPRET = False


def _split_dot(x, m):
    xh = x.astype(jnp.bfloat16).astype(jnp.float32)
    xl = x - xh
    return (jnp.dot(xh, m, preferred_element_type=jnp.float32)
            + jnp.dot(xl, m, preferred_element_type=jnp.float32))


def _ln(x, g, beta):
    d = x.shape[-1]
    io = jax.lax.broadcasted_iota(jnp.int32, (d, d), 0)
    ones_d = jnp.where(io >= 0, np.float32(1.0), np.float32(0.0))
    inv = np.float32(1.0 / d)
    mu = _split_dot(x, ones_d) * inv
    ex2 = _split_dot(x * x, ones_d) * inv
    rs = jax.lax.rsqrt(ex2 - mu * mu + 1e-5)
    return (x - mu) * rs * g + beta


def _gelu(x):
    return x * 0.5 * (1.0 + jax.lax.erf(x / np.sqrt(2.0).astype(np.float32)))


def _stem_body(x_ref, w1, b1, g1, be1, w2, b2, g2, be2, out_ref):
    x6 = x_ref[...]
    x = jnp.concatenate(
        [x6, jnp.zeros((x6.shape[0], 2), jnp.float32)], axis=1)
    h = jnp.dot(x, w1[...], preferred_element_type=jnp.float32) + b1[...]
    h = _gelu(_ln(h, g1[...], be1[...]))
    h = jnp.dot(h, w2[...], preferred_element_type=jnp.float32) + b2[...]
    h = _gelu(_ln(h, g2[...], be2[...]))
    xyz = x[:, 0:3]
    pad = jnp.zeros((x.shape[0], _TBL_D - _STEM - 3), jnp.float32)
    out_ref[...] = jnp.concatenate([h, xyz, pad], axis=1)


def _run_stem(xpad, sp):
    blk = 8192
    grid = (_B * _N // blk,)
    full = lambda a: pl.BlockSpec(a.shape, lambda i: (0,) * a.ndim)
    params = []
    specs = [pl.BlockSpec((blk, 6), lambda i: (i, 0))]
    for layer in sp:
        for arr in layer:
            params.append(arr)
            specs.append(full(arr))
    return pl.pallas_call(
        _stem_body,
        grid=grid,
        in_specs=specs,
        out_specs=pl.BlockSpec((blk, _TBL_D), lambda i: (i, 0)),
        out_shape=jax.ShapeDtypeStruct((_B * _N, _TBL_D), jnp.float32),
        interpret=_INTERPRET,
    )(xpad, *params)


def _fps_body(x_ref, y_ref, z_ref, cidx_ref, cen_ref, dist_ref):
    shp = (_B, _N // 128, 128)
    ax = (1, 2)
    flat = (jax.lax.broadcasted_iota(jnp.int32, shp, 1) * 128
            + jax.lax.broadcasted_iota(jnp.int32, shp, 2))
    ii64 = jax.lax.broadcasted_iota(jnp.int32, (_B, _S), 1)
    cc_i = jax.lax.broadcasted_iota(jnp.int32, (_B, 3, _S), 2)
    dist_ref[...] = jnp.full(shp, 1e10, jnp.float32)

    def body(i, carry):
        far, ci, cc = carry
        ci = jnp.where(ii64 == i, far[:, :, 0], ci)
        sel = flat == far
        cx = jnp.sum(jnp.where(sel, x_ref[...], 0.0), axis=ax, keepdims=True)
        cy = jnp.sum(jnp.where(sel, y_ref[...], 0.0), axis=ax, keepdims=True)
        cz = jnp.sum(jnp.where(sel, z_ref[...], 0.0), axis=ax, keepdims=True)
        coords = jnp.concatenate(
            [cx[:, :, 0], cy[:, :, 0], cz[:, :, 0]],
            axis=1)[:, :, None]
        cc = jnp.where(cc_i == i, coords, cc)
        dx = x_ref[...] - cx
        dy = y_ref[...] - cy
        dz = z_ref[...] - cz
        d = dx * dx + dy * dy + dz * dz
        dn = jnp.minimum(dist_ref[...], d)
        dist_ref[...] = dn
        m = jnp.max(dn, axis=ax, keepdims=True)
        far = jnp.min(jnp.where(dn == m, flat, jnp.int32(_N)),
                      axis=ax, keepdims=True)
        return far, ci, cc

    far0 = jnp.zeros((_B, 1, 1), jnp.int32)
    ci0 = jnp.zeros((_B, _S), jnp.int32)
    cc0 = jnp.zeros((_B, 3, _S), jnp.float32)
    _, ci, cc = jax.lax.fori_loop(0, _S, body, (far0, ci0, cc0))
    cidx_ref[...] = ci
    cen_ref[...] = cc


def _run_fps(x3, y3, z3):
    return pl.pallas_call(
        _fps_body,
        out_shape=(jax.ShapeDtypeStruct((_B, _S), jnp.int32),
                   jax.ShapeDtypeStruct((_B, 3, _S), jnp.float32)),
        scratch_shapes=[pltpu.VMEM((_B, _N // 128, 128), jnp.float32)],
        interpret=_INTERPRET,
    )(x3, y3, z3)


def _ball_body(x_ref, y_ref, z_ref, cx_ref, cy_ref, cz_ref, cidx_ref,
               oall_ref, acc1, acc2, cnt1, cnt2):
    b = pl.program_id(0)
    j = pl.program_id(1)

    @pl.when(j == 0)
    def _():
        acc1[...] = jnp.zeros_like(acc1)
        acc2[...] = jnp.zeros_like(acc2)
        cnt1[...] = jnp.zeros_like(cnt1)
        cnt2[...] = jnp.zeros_like(cnt2)

    dx = cx_ref[0] - x_ref[0, 0]
    dy = cy_ref[0] - y_ref[0, 0]
    dz = cz_ref[0] - z_ref[0, 0]
    d2 = dx * dx + dy * dy + dz * dz
    jg = (j * _C + jax.lax.broadcasted_iota(jnp.int32, (1, _C), 1))
    jgb = jnp.broadcast_to(jg, (_S, _C))

    in1 = d2 <= np.float32(_RADII[0] * _RADII[0])
    in2 = d2 <= np.float32(_RADII[1] * _RADII[1])
    mp = (jnp.where(in1, 1 << 13, 0) + jnp.where(in2, 1, 0))
    sh = 1
    while sh < _C:
        z = jnp.zeros((_S, sh), jnp.int32)
        mp = mp + jnp.concatenate([z, mp[:, : _C - sh]], axis=1)
        sh *= 2
    for inm, rank, K, acc, cnt in (
            (in1, mp >> 13, _KS[0], acc1, cnt1),
            (in2, mp & 8191, _KS[1], acc2, cnt2)):
        cold = cnt[...]
        rr = jnp.where(inm, rank + cold, 0)
        contribs = [jnp.sum(jnp.where(rr == k + 1, jgb, 0),
                            axis=1, keepdims=True) for k in range(K)]
        acc[...] += jnp.concatenate(contribs, axis=1)
        cnt[...] = cold + rank[:, _C - 1:_C]

    @pl.when(j == _NCH - 1)
    def _():
        off = b * _N
        for K, acc, cnt, lo in ((_KS[0], acc1, cnt1, 0),
                                (_KS[1], acc2, cnt2, _KS[0])):
            ids = acc[...]
            kk = jax.lax.broadcasted_iota(jnp.int32, (_S, K), 1)
            padded = jnp.where(kk < cnt[...], ids, ids[:, 0:1]) + off
            oall_ref[0, lo:lo + K, :] = jnp.transpose(padded, (1, 0))
        oall_ref[0, 48:49, :] = cidx_ref[0] + off


def _run_ball(x2, y2, z2, cenx, ceny, cenz, cidx3):
    cspec = pl.BlockSpec((1, _S, 1), lambda b, j: (b, 0, 0))
    return pl.pallas_call(
        _ball_body,
        grid=(_B, _NCH),
        in_specs=[
            pl.BlockSpec((1, 1, 1, _C), lambda b, j: (b, j, 0, 0)),
            pl.BlockSpec((1, 1, 1, _C), lambda b, j: (b, j, 0, 0)),
            pl.BlockSpec((1, 1, 1, _C), lambda b, j: (b, j, 0, 0)),
            cspec, cspec, cspec,
            pl.BlockSpec((1, 1, _S), lambda b, j: (b, 0, 0)),
        ],
        out_specs=pl.BlockSpec((1, 49, _S), lambda b, j: (b, 0, 0)),
        out_shape=jax.ShapeDtypeStruct((_B, 49, _S), jnp.int32),
        scratch_shapes=[
            pltpu.VMEM((_S, _KS[0]), jnp.int32),
            pltpu.VMEM((_S, _KS[1]), jnp.int32),
            pltpu.VMEM((_S, 1), jnp.int32),
            pltpu.VMEM((_S, 1), jnp.int32),
        ],
        interpret=_INTERPRET,
    )(x2, y2, z2, cenx, ceny, cenz, cidx3)


def _gather_rows(table, gidx):
    nw = 32
    bpw = _G // nw
    mesh = plsc.VectorSubcoreMesh(core_axis_name="c", subcore_axis_name="s")

    @functools.partial(
        pl.kernel,
        mesh=mesh,
        out_type=jax.ShapeDtypeStruct((_G, _TBL_D), jnp.float32),
        scratch_types=[
            pltpu.VMEM((bpw,), jnp.int32),
            pltpu.VMEM((bpw, _TBL_D), jnp.float32),
            pltpu.SemaphoreType.DMA,
        ],
    )
    def k(table_hbm, idx_hbm, out_hbm, idx_v, rows_v, sem):
        wid = jax.lax.axis_index("s") * 2 + jax.lax.axis_index("c")
        base = wid * bpw
        pltpu.sync_copy(idx_hbm.at[pl.ds(base, bpw)], idx_v)
        pltpu.async_copy(table_hbm.at[idx_v], rows_v, sem).wait()
        pltpu.sync_copy(rows_v, out_hbm.at[pl.ds(base, bpw)])

    return k(table, gidx)


def _sin3(x, dim):
    d_axis = dim // 3
    half = d_axis // 2
    scale = np.float32(-np.log(10000.0) / max(half - 1, 1))
    col = jax.lax.broadcasted_iota(jnp.int32, (3, dim), 1)
    row = jax.lax.broadcasted_iota(jnp.int32, (3, dim), 0)
    freqs = jnp.exp((col % half).astype(jnp.float32) * scale)
    proj = jnp.where(col // d_axis == row, freqs, np.float32(0.0))
    ph = proj.astype(jnp.bfloat16).astype(jnp.float32)
    ang = _split_dot(x, ph) + jnp.dot(
        x, proj - ph, preferred_element_type=jnp.float32)
    colf = jax.lax.broadcasted_iota(jnp.int32, (x.shape[0], dim), 1)
    off = jnp.where((colf % d_axis) >= half, np.float32(np.pi / 2),
                    np.float32(0.0))
    return jnp.sin(ang + off)


def _token_body(g_ref, ce_ref,
                e10w, e10b, e10g, e10be, e11w, e11b, e11g, e11be,
                e20w, e20b, e20g, e20be, e21w, e21b, e21g, e21be,
                p0w, p0b, p0g, p0be, p1w, p1b, p1g, p1be,
                tok_ref):
    gg = g_ref[0]
    cen = ce_ref[0]
    cf = gg[3072:3136, 0:_STEM]

    def scale(lo, hi, K, w0, b0, g0, be0, w1, b1, g1, be1):
        gf = gg[lo:hi, 0:_STEM]
        gx = gg[lo:hi, _STEM:_STEM + 3]
        rel = gx - jnp.concatenate([cen] * K, axis=0)
        rpe = _sin3(rel, 24)
        pad = jnp.zeros((hi - lo, 5), jnp.float32)
        gi = jnp.concatenate([gf, rel, rpe, pad], axis=1)
        h = jnp.dot(gi, w0[...], preferred_element_type=jnp.float32) + b0[...]
        h = _gelu(_ln(h, g0[...], be0[...]))
        h = jnp.dot(h, w1[...], preferred_element_type=jnp.float32) + b1[...]
        h = _ln(h, g1[...], be1[...])
        mx = h[0:_S]
        for kk in range(1, K):
            mx = jnp.maximum(mx, h[kk * _S:(kk + 1) * _S])
        return mx

    lo1, hi1 = 0, _S * _KS[0]
    lo2, hi2 = hi1, hi1 + _S * _KS[1]
    mx1 = scale(lo1, hi1, _KS[0], e10w, e10b, e10g, e10be,
                e11w, e11b, e11g, e11be)
    mx2 = scale(lo2, hi2, _KS[1], e20w, e20b, e20g, e20be,
                e21w, e21b, e21g, e21be)
    cpos = _sin3(cen, 96)
    t = jnp.concatenate([cf, mx1, mx2, cpos], axis=1)
    t = jnp.dot(t, p0w[...], preferred_element_type=jnp.float32) + p0b[...]
    t = _gelu(_ln(t, p0g[...], p0be[...]))
    t = jnp.dot(t, p1w[...], preferred_element_type=jnp.float32) + p1b[...]
    t = _ln(t, p1g[...], p1be[...])
    tok_ref[...] = t[None]


def _run_token(g3, cen, flat_params):
    full = lambda a: pl.BlockSpec(a.shape, lambda b: (0,) * a.ndim)
    specs = [
        pl.BlockSpec((1, _ROWS_PER_B, _TBL_D), lambda b: (b, 0, 0)),
        pl.BlockSpec((1, _S, 3), lambda b: (b, 0, 0)),
    ] + [full(a) for a in flat_params]
    return pl.pallas_call(
        _token_body,
        grid=(_B,),
        in_specs=specs,
        out_specs=pl.BlockSpec((1, _S, _TOK), lambda b: (b, 0, 0)),
        out_shape=jax.ShapeDtypeStruct((_B, _S, _TOK), jnp.float32),
        interpret=_INTERPRET,
    )(g3, cen, *flat_params)


def _prep_layer(p, pad_rows=None):
    w = p['W']
    if pad_rows is not None and w.shape[0] < pad_rows:
        w = jnp.pad(w, ((0, pad_rows - w.shape[0]), (0, 0)))
    return (w, p['b'][None, :], p['g'][None, :], p['beta'][None, :])


def kernel(pointcloud, params):
    pc = pointcloud.reshape(_B * _N, 6)

    x2 = pointcloud[..., 0]
    y2 = pointcloud[..., 1]
    z2 = pointcloud[..., 2]
    x4 = x2.reshape(_B, _NCH, 1, _C)
    y4 = y2.reshape(_B, _NCH, 1, _C)
    z4 = z2.reshape(_B, _NCH, 1, _C)

    stem_params = [_prep_layer(params['stem'][0], pad_rows=8),
                   _prep_layer(params['stem'][1])]
    table = _run_stem(pc, stem_params)

    cidx, cc = _run_fps(x2.reshape(_B, _N // 128, 128),
                        y2.reshape(_B, _N // 128, 128),
                        z2.reshape(_B, _N // 128, 128))
    cenx = cc[:, 0, :, None]
    ceny = cc[:, 1, :, None]
    cenz = cc[:, 2, :, None]
    centers = jnp.transpose(cc, (0, 2, 1))

    oall = _run_ball(x4, y4, z4, cenx, ceny, cenz, cidx[:, None, :])

    gidx = oall.reshape(_G)

    rows = _gather_rows(table, gidx)
    g3 = rows.reshape(_B, _ROWS_PER_B, _TBL_D)

    flat_params = []
    for layer_i, layer in enumerate(params['enc'][0] + params['enc'][1]):
        flat_params.extend(_prep_layer(layer, pad_rows=96 if layer_i % 2 == 0 else None))
    for layer in params['proj']:
        flat_params.extend(_prep_layer(layer))

    tok = _run_token(g3, centers, flat_params)
    return tok, centers


# speedup vs baseline: 1.7251x; 1.0005x over previous
import functools

import jax
import jax.numpy as jnp
import numpy as np
from jax.experimental import pallas as pl
from jax.experimental.pallas import tpu as pltpu
from jax.experimental.pallas import tpu_sc as plsc

_B, _N = 8, 32768
_S = 64
_STEM = 64
_TOK = 128
_RADII = (0.04, 0.08)
_KS = (16, 32)
_TBL_D = 128
_ROWS_PER_B = _S + _S * _KS[0] + _S * _KS[1]
_G = _B * _ROWS_PER_B
_C = 8192
_NCH = _N // _C

_INTERPRET = False


def _split_dot(x, m):
    xh = x.astype(jnp.bfloat16).astype(jnp.float32)
    xl = x - xh
    return (jnp.dot(xh, m, preferred_element_type=jnp.float32)
            + jnp.dot(xl, m, preferred_element_type=jnp.float32))


def _ln(x, g, beta):
    d = x.shape[-1]
    io = jax.lax.broadcasted_iota(jnp.int32, (d, d), 0)
    ones_d = jnp.where(io >= 0, np.float32(1.0), np.float32(0.0))
    inv = np.float32(1.0 / d)
    mu = _split_dot(x, ones_d) * inv
    ex2 = _split_dot(x * x, ones_d) * inv
    rs = jax.lax.rsqrt(ex2 - mu * mu + 1e-5)
    return (x - mu) * rs * g + beta


def _gelu(x):
    return x * 0.5 * (1.0 + jax.lax.erf(x / np.sqrt(2.0).astype(np.float32)))


def _stem_body(x_ref, w1, b1, g1, be1, w2, b2, g2, be2, out_ref):
    x6 = x_ref[...]
    x = jnp.concatenate(
        [x6, jnp.zeros((x6.shape[0], 2), jnp.float32)], axis=1)
    h = jnp.dot(x, w1[...], preferred_element_type=jnp.float32) + b1[...]
    h = _gelu(_ln(h, g1[...], be1[...]))
    h = jnp.dot(h, w2[...], preferred_element_type=jnp.float32) + b2[...]
    h = _gelu(_ln(h, g2[...], be2[...]))
    xyz = x[:, 0:3]
    pad = jnp.zeros((x.shape[0], _TBL_D - _STEM - 3), jnp.float32)
    out_ref[...] = jnp.concatenate([h, xyz, pad], axis=1)


def _run_stem(xpad, sp):
    blk = 8192
    grid = (_B * _N // blk,)
    full = lambda a: pl.BlockSpec(a.shape, lambda i: (0,) * a.ndim)
    params = []
    specs = [pl.BlockSpec((blk, 6), lambda i: (i, 0))]
    for layer in sp:
        for arr in layer:
            params.append(arr)
            specs.append(full(arr))
    return pl.pallas_call(
        _stem_body,
        grid=grid,
        in_specs=specs,
        out_specs=pl.BlockSpec((blk, _TBL_D), lambda i: (i, 0)),
        out_shape=jax.ShapeDtypeStruct((_B * _N, _TBL_D), jnp.float32),
        interpret=_INTERPRET,
    )(xpad, *params)


def _fps_body(x_ref, y_ref, z_ref, cidx_ref, cen_ref, dist_ref):
    shp = (_B, _N // 128, 128)
    ax = (1, 2)
    flat = (jax.lax.broadcasted_iota(jnp.int32, shp, 1) * 128
            + jax.lax.broadcasted_iota(jnp.int32, shp, 2))
    ii64 = jax.lax.broadcasted_iota(jnp.int32, (_B, _S), 1)
    cc_i = jax.lax.broadcasted_iota(jnp.int32, (_B, 3, _S), 2)
    dist_ref[...] = jnp.full(shp, 1e10, jnp.float32)

    def body(i, carry):
        far, ci, cc = carry
        ci = jnp.where(ii64 == i, far[:, :, 0], ci)
        sel = flat == far
        cx = jnp.sum(jnp.where(sel, x_ref[...], 0.0), axis=ax, keepdims=True)
        cy = jnp.sum(jnp.where(sel, y_ref[...], 0.0), axis=ax, keepdims=True)
        cz = jnp.sum(jnp.where(sel, z_ref[...], 0.0), axis=ax, keepdims=True)
        coords = jnp.concatenate(
            [cx[:, :, 0], cy[:, :, 0], cz[:, :, 0]],
            axis=1)[:, :, None]
        cc = jnp.where(cc_i == i, coords, cc)
        dx = x_ref[...] - cx
        dy = y_ref[...] - cy
        dz = z_ref[...] - cz
        d = dx * dx + dy * dy + dz * dz
        dn = jnp.minimum(dist_ref[...], d)
        dist_ref[...] = dn
        m = jnp.max(dn, axis=ax, keepdims=True)
        far = jnp.min(jnp.where(dn == m, flat, jnp.int32(_N)),
                      axis=ax, keepdims=True)
        return far, ci, cc

    far0 = jnp.zeros((_B, 1, 1), jnp.int32)
    ci0 = jnp.zeros((_B, _S), jnp.int32)
    cc0 = jnp.zeros((_B, 3, _S), jnp.float32)
    _, ci, cc = jax.lax.fori_loop(0, _S, body, (far0, ci0, cc0))
    cidx_ref[...] = ci
    cen_ref[...] = cc


def _run_fps(x3, y3, z3):
    return pl.pallas_call(
        _fps_body,
        out_shape=(jax.ShapeDtypeStruct((_B, _S), jnp.int32),
                   jax.ShapeDtypeStruct((_B, 3, _S), jnp.float32)),
        scratch_shapes=[pltpu.VMEM((_B, _N // 128, 128), jnp.float32)],
        interpret=_INTERPRET,
    )(x3, y3, z3)


def _ball_body(x_ref, y_ref, z_ref, cx_ref, cy_ref, cz_ref, cidx_ref,
               oall_ref, acc1, acc2, cnt1, cnt2):
    b = pl.program_id(0)
    j = pl.program_id(1)

    @pl.when(j == 0)
    def _():
        acc1[...] = jnp.zeros_like(acc1)
        acc2[...] = jnp.zeros_like(acc2)
        cnt1[...] = jnp.zeros_like(cnt1)
        cnt2[...] = jnp.zeros_like(cnt2)

    dx = cx_ref[0] - x_ref[0, 0]
    dy = cy_ref[0] - y_ref[0, 0]
    dz = cz_ref[0] - z_ref[0, 0]
    d2 = dx * dx + dy * dy + dz * dz
    jg = (j * _C + jax.lax.broadcasted_iota(jnp.int32, (1, _C), 1))
    jgb = jnp.broadcast_to(jg, (_S, _C))

    in1 = d2 <= np.float32(_RADII[0] * _RADII[0])
    in2 = d2 <= np.float32(_RADII[1] * _RADII[1])
    mp = (jnp.where(in1, 1 << 14, 0) + jnp.where(in2, 1, 0))
    sh = 1
    while sh < _C:
        z = jnp.zeros((_S, sh), jnp.int32)
        mp = mp + jnp.concatenate([z, mp[:, : _C - sh]], axis=1)
        sh *= 2
    for inm, rank, K, acc, cnt in (
            (in1, mp >> 14, _KS[0], acc1, cnt1),
            (in2, mp & 16383, _KS[1], acc2, cnt2)):
        cold = cnt[...]
        rr = jnp.where(inm, rank + cold, 0)
        contribs = [jnp.sum(jnp.where(rr == k + 1, jgb, 0),
                            axis=1, keepdims=True) for k in range(K)]
        acc[...] += jnp.concatenate(contribs, axis=1)
        cnt[...] = cold + rank[:, _C - 1:_C]

    @pl.when(j == _NCH - 1)
    def _():
        off = b * _N
        for K, acc, cnt, lo in ((_KS[0], acc1, cnt1, 0),
                                (_KS[1], acc2, cnt2, _KS[0])):
            ids = acc[...]
            kk = jax.lax.broadcasted_iota(jnp.int32, (_S, K), 1)
            padded = jnp.where(kk < cnt[...], ids, ids[:, 0:1]) + off
            oall_ref[0, lo:lo + K, :] = jnp.transpose(padded, (1, 0))
        oall_ref[0, 48:49, :] = cidx_ref[0] + off


def _run_ball(x2, y2, z2, cenx, ceny, cenz, cidx3):
    cspec = pl.BlockSpec((1, _S, 1), lambda b, j: (b, 0, 0))
    return pl.pallas_call(
        _ball_body,
        grid=(_B, _NCH),
        in_specs=[
            pl.BlockSpec((1, 1, 1, _C), lambda b, j: (b, j, 0, 0)),
            pl.BlockSpec((1, 1, 1, _C), lambda b, j: (b, j, 0, 0)),
            pl.BlockSpec((1, 1, 1, _C), lambda b, j: (b, j, 0, 0)),
            cspec, cspec, cspec,
            pl.BlockSpec((1, 1, _S), lambda b, j: (b, 0, 0)),
        ],
        out_specs=pl.BlockSpec((1, 49, _S), lambda b, j: (b, 0, 0)),
        out_shape=jax.ShapeDtypeStruct((_B, 49, _S), jnp.int32),
        scratch_shapes=[
            pltpu.VMEM((_S, _KS[0]), jnp.int32),
            pltpu.VMEM((_S, _KS[1]), jnp.int32),
            pltpu.VMEM((_S, 1), jnp.int32),
            pltpu.VMEM((_S, 1), jnp.int32),
        ],
        interpret=_INTERPRET,
    )(x2, y2, z2, cenx, ceny, cenz, cidx3)


def _gather_rows(table, gidx):
    nw = 32
    bpw = _G // nw
    mesh = plsc.VectorSubcoreMesh(core_axis_name="c", subcore_axis_name="s")

    @functools.partial(
        pl.kernel,
        mesh=mesh,
        out_type=jax.ShapeDtypeStruct((_G, _TBL_D), jnp.float32),
        scratch_types=[
            pltpu.VMEM((bpw,), jnp.int32),
            pltpu.VMEM((bpw, _TBL_D), jnp.float32),
            pltpu.SemaphoreType.DMA,
        ],
    )
    def k(table_hbm, idx_hbm, out_hbm, idx_v, rows_v, sem):
        wid = jax.lax.axis_index("s") * 2 + jax.lax.axis_index("c")
        base = wid * bpw
        pltpu.sync_copy(idx_hbm.at[pl.ds(base, bpw)], idx_v)
        pltpu.async_copy(table_hbm.at[idx_v], rows_v, sem).wait()
        pltpu.sync_copy(rows_v, out_hbm.at[pl.ds(base, bpw)])

    return k(table, gidx)


def _sin3(x, dim):
    d_axis = dim // 3
    half = d_axis // 2
    scale = np.float32(-np.log(10000.0) / max(half - 1, 1))
    col = jax.lax.broadcasted_iota(jnp.int32, (3, dim), 1)
    row = jax.lax.broadcasted_iota(jnp.int32, (3, dim), 0)
    freqs = jnp.exp((col % half).astype(jnp.float32) * scale)
    proj = jnp.where(col // d_axis == row, freqs, np.float32(0.0))
    ph = proj.astype(jnp.bfloat16).astype(jnp.float32)
    ang = _split_dot(x, ph) + jnp.dot(
        x, proj - ph, preferred_element_type=jnp.float32)
    colf = jax.lax.broadcasted_iota(jnp.int32, (x.shape[0], dim), 1)
    off = jnp.where((colf % d_axis) >= half, np.float32(np.pi / 2),
                    np.float32(0.0))
    return jnp.sin(ang + off)


def _token_body(g_ref, ce_ref,
                e10w, e10b, e10g, e10be, e11w, e11b, e11g, e11be,
                e20w, e20b, e20g, e20be, e21w, e21b, e21g, e21be,
                p0w, p0b, p0g, p0be, p1w, p1b, p1g, p1be,
                tok_ref):
    gg = g_ref[0]
    cen = ce_ref[0]
    cf = gg[3072:3136, 0:_STEM]

    def scale(lo, hi, K, w0, b0, g0, be0, w1, b1, g1, be1):
        gf = gg[lo:hi, 0:_STEM]
        gx = gg[lo:hi, _STEM:_STEM + 3]
        rel = gx - jnp.concatenate([cen] * K, axis=0)
        rpe = _sin3(rel, 24)
        pad = jnp.zeros((hi - lo, 5), jnp.float32)
        gi = jnp.concatenate([gf, rel, rpe, pad], axis=1)
        h = jnp.dot(gi, w0[...], preferred_element_type=jnp.float32) + b0[...]
        h = _gelu(_ln(h, g0[...], be0[...]))
        h = jnp.dot(h, w1[...], preferred_element_type=jnp.float32) + b1[...]
        h = _ln(h, g1[...], be1[...])
        mx = h[0:_S]
        for kk in range(1, K):
            mx = jnp.maximum(mx, h[kk * _S:(kk + 1) * _S])
        return mx

    lo1, hi1 = 0, _S * _KS[0]
    lo2, hi2 = hi1, hi1 + _S * _KS[1]
    mx1 = scale(lo1, hi1, _KS[0], e10w, e10b, e10g, e10be,
                e11w, e11b, e11g, e11be)
    mx2 = scale(lo2, hi2, _KS[1], e20w, e20b, e20g, e20be,
                e21w, e21b, e21g, e21be)
    cpos = _sin3(cen, 96)
    t = jnp.concatenate([cf, mx1, mx2, cpos], axis=1)
    t = jnp.dot(t, p0w[...], preferred_element_type=jnp.float32) + p0b[...]
    t = _gelu(_ln(t, p0g[...], p0be[...]))
    t = jnp.dot(t, p1w[...], preferred_element_type=jnp.float32) + p1b[...]
    t = _ln(t, p1g[...], p1be[...])
    tok_ref[...] = t[None]


def _run_token(g3, cen, flat_params):
    full = lambda a: pl.BlockSpec(a.shape, lambda b: (0,) * a.ndim)
    specs = [
        pl.BlockSpec((1, _ROWS_PER_B, _TBL_D), lambda b: (b, 0, 0)),
        pl.BlockSpec((1, _S, 3), lambda b: (b, 0, 0)),
    ] + [full(a) for a in flat_params]
    return pl.pallas_call(
        _token_body,
        grid=(_B,),
        in_specs=specs,
        out_specs=pl.BlockSpec((1, _S, _TOK), lambda b: (b, 0, 0)),
        out_shape=jax.ShapeDtypeStruct((_B, _S, _TOK), jnp.float32),
        interpret=_INTERPRET,
    )(g3, cen, *flat_params)


def _prep_layer(p, pad_rows=None):
    w = p['W']
    if pad_rows is not None and w.shape[0] < pad_rows:
        w = jnp.pad(w, ((0, pad_rows - w.shape[0]), (0, 0)))
    return (w, p['b'][None, :], p['g'][None, :], p['beta'][None, :])


def kernel(pointcloud, params):
    pc = pointcloud.reshape(_B * _N, 6)

    x2 = pointcloud[..., 0]
    y2 = pointcloud[..., 1]
    z2 = pointcloud[..., 2]
    x4 = x2.reshape(_B, _NCH, 1, _C)
    y4 = y2.reshape(_B, _NCH, 1, _C)
    z4 = z2.reshape(_B, _NCH, 1, _C)

    stem_params = [_prep_layer(params['stem'][0], pad_rows=8),
                   _prep_layer(params['stem'][1])]
    table = _run_stem(pc, stem_params)

    cidx, cc = _run_fps(x2.reshape(_B, _N // 128, 128),
                        y2.reshape(_B, _N // 128, 128),
                        z2.reshape(_B, _N // 128, 128))
    cenx = cc[:, 0, :, None]
    ceny = cc[:, 1, :, None]
    cenz = cc[:, 2, :, None]
    centers = jnp.transpose(cc, (0, 2, 1))

    oall = _run_ball(x4, y4, z4, cenx, ceny, cenz, cidx[:, None, :])

    gidx = oall.reshape(_G)

    rows = _gather_rows(table, gidx)
    g3 = rows.reshape(_B, _ROWS_PER_B, _TBL_D)

    flat_params = []
    for layer_i, layer in enumerate(params['enc'][0] + params['enc'][1]):
        flat_params.extend(_prep_layer(layer, pad_rows=96 if layer_i % 2 == 0 else None))
    for layer in params['proj']:
        flat_params.extend(_prep_layer(layer))

    tok = _run_token(g3, centers, flat_params)
    return tok, centers
